# baseline (speedup 1.0000x reference)
import jax
import jax.numpy as jnp
from jax.experimental import pallas as pl
from jax.experimental.pallas import tpu as pltpu


def _mse_kernel(p_ref, t_ref, m_ref, out_ref, acc_ref):
    i = pl.program_id(0)

    @pl.when(i == 0)
    def _init():
        acc_ref[0] = 0.0
        acc_ref[1] = 0.0

    d = p_ref[...] - t_ref[...]
    keep = jnp.logical_not(m_ref[...])
    sq = jnp.where(keep, d * d, 0.0)
    acc_ref[0] += jnp.sum(sq)
    acc_ref[1] += jnp.sum(keep.astype(jnp.float32))

    @pl.when(i == pl.num_programs(0) - 1)
    def _fin():
        out_ref[...] = jnp.full((1, 1), acc_ref[0] / acc_ref[1], jnp.float32)


def kernel(preds, target, mask):
    B, T = preds.shape
    BR = 256
    out = pl.pallas_call(
        _mse_kernel,
        grid=(B // BR,),
        in_specs=[
            pl.BlockSpec((BR, T), lambda i: (i, 0)),
            pl.BlockSpec((BR, T), lambda i: (i, 0)),
            pl.BlockSpec((BR, T), lambda i: (i, 0)),
        ],
        out_specs=pl.BlockSpec((1, 1), lambda i: (0, 0)),
        out_shape=jax.ShapeDtypeStruct((1, 1), jnp.float32),
        scratch_shapes=[pltpu.SMEM((2,), jnp.float32)],
    )(preds, target, mask)
    return out[0, 0]
